# baseline (speedup 1.0000x reference)
import functools

import jax
import jax.numpy as jnp
from jax import lax
from jax.experimental import pallas as pl
from jax.experimental.pallas import tpu as pltpu
from jax.experimental.pallas import tpu_sc as plsc

F32 = jnp.float32
I32 = jnp.int32
NHEADS = 4
NHID = 32
DD = NHEADS * NHID
KNODE = 4
NCORE = 2
NSUB = 16
NWORK = NCORE * NSUB


def _feat_body(x_ref, w_ref, asrc_ref, adst_ref, s8s_ref, s8d_ref, h_ref, atab_ref):
    x = x_ref[0]
    w = w_ref[0]
    h = jnp.dot(x, w, preferred_element_type=F32)
    h_ref[0] = h
    asrc = asrc_ref[0]
    adst = adst_ref[0]
    atab_ref[0] = (jnp.dot(h * asrc, s8s_ref[...],
                           preferred_element_type=F32)
                   + jnp.dot(h * adst, s8d_ref[...],
                             preferred_element_type=F32))


def _feat_tc(ng, n, fin, x, w, asrc, adst, s8s, s8d):
    return pl.pallas_call(
        _feat_body,
        grid=(ng,),
        in_specs=[
            pl.BlockSpec((1, n, fin), lambda g: (g, 0, 0)),
            pl.BlockSpec((1, fin, DD), lambda g: (g, 0, 0)),
            pl.BlockSpec((1, 1, DD), lambda g: (g, 0, 0)),
            pl.BlockSpec((1, 1, DD), lambda g: (g, 0, 0)),
            pl.BlockSpec((DD, 8), lambda g: (0, 0)),
            pl.BlockSpec((DD, 8), lambda g: (0, 0)),
        ],
        out_specs=[
            pl.BlockSpec((1, n, DD), lambda g: (g, 0, 0)),
            pl.BlockSpec((1, n, 8), lambda g: (g, 0, 0)),
        ],
        out_shape=[
            jax.ShapeDtypeStruct((ng, n, DD), F32),
            jax.ShapeDtypeStruct((ng, n, 8), F32),
        ],
    )(x, w, asrc, adst, s8s, s8d)


def _make_gat_sc(ng, n, e):
    ew = e // NWORK
    nch = ew // 128
    nvr = ew // 16
    n16 = n // NSUB
    mesh = plsc.VectorSubcoreMesh(core_axis_name="c", subcore_axis_name="s")

    @functools.partial(
        pl.kernel,
        mesh=mesh,
        compiler_params=pltpu.CompilerParams(needs_layout_passes=False),
        out_type=[
            jax.ShapeDtypeStruct((ng, NCORE, n, DD), F32),
            jax.ShapeDtypeStruct((ng, NWORK * NHEADS * n), F32),
        ],
        scratch_types=[
            pltpu.VMEM((ew,), I32),
            pltpu.VMEM((ew,), I32),
            pltpu.VMEM((n * 8,), F32),
            pltpu.VMEM((NHEADS * ew,), F32),
            pltpu.VMEM((NHEADS * n,), F32),
            pltpu.VMEM((nch, 128), I32),
            pltpu.VMEM((nch, 128), I32),
            pltpu.VMEM((ew, DD), F32),
            pltpu.VMEM((n16, DD), F32),
            pltpu.VMEM_SHARED((n, DD), F32),
            pltpu.SemaphoreType.DMA,
            pltpu.SemaphoreType.DMA,
        ],
    )
    def gat(edges_hbm, h_hbm, atab_hbm, convraw_hbm, dpart_hbm,
            src_v, dst_v, atab_v, ee_v, den_v, gidx_v, didx_v, rows_v,
            zero_v, shout, gsem, ssem):
        cid = lax.axis_index("c")
        sid = lax.axis_index("s")
        wid = cid * NSUB + sid
        base = wid * ew
        zv16 = jnp.zeros((16,), F32)

        def zrow(r, _):
            for c in range(DD // 16):
                zero_v[r, pl.ds(c * 16, 16)] = zv16
            return 0
        lax.fori_loop(0, n16, zrow, 0)

        def gbody(g, _):
            pltpu.sync_copy(edges_hbm.at[g, 0, pl.ds(base, ew)], src_v)
            pltpu.sync_copy(edges_hbm.at[g, 1, pl.ds(base, ew)], dst_v)

            @plsc.parallel_loop(0, nvr, step=1, unroll=4)
            def _ibody(i):
                srcv = src_v[pl.ds(i * 16, 16)]
                gidx_v[i // 8, pl.ds((i % 8) * 16, 16)] = srcv + g * n
                didx_v[i // 8, pl.ds((i % 8) * 16, 16)] = dst_v[pl.ds(i * 16, 16)]

            gdesc = [pltpu.async_copy(h_hbm.at[gidx_v.at[j]],
                                      rows_v.at[pl.ds(j * 128, 128)], gsem)
                     for j in range(nch)]

            pltpu.sync_copy(atab_hbm.at[g], atab_v)
            pltpu.sync_copy(zero_v, shout.at[pl.ds(sid * n16, n16)])

            def dzero(i, _):
                den_v[pl.ds(i * 16, 16)] = zv16
                return 0
            lax.fori_loop(0, NHEADS * n // 16, dzero, 0)

            def ebody(i, _):
                srcv = src_v[pl.ds(i * 16, 16)]
                dstv = dst_v[pl.ds(i * 16, 16)]
                for hd in range(NHEADS):
                    a_s = plsc.load_gather(atab_v, [srcv * 8 + hd])
                    a_d = plsc.load_gather(atab_v, [dstv * 8 + (hd + 4)])
                    ev = a_s + a_d
                    ev = jnp.where(ev >= 0.0, ev, ev * 0.2)
                    eev = jnp.exp(ev)
                    ee_v[pl.ds(hd * ew + i * 16, 16)] = eev
                    plsc.addupdate_scatter(den_v, [dstv + hd * n], eev)
                return 0
            lax.fori_loop(0, nvr, ebody, 0)

            pltpu.sync_copy(den_v,
                            dpart_hbm.at[g, pl.ds(wid * (NHEADS * n),
                                                  NHEADS * n)])
            plsc.subcore_barrier()

            sdesc = []
            for j in range(nch):
                gdesc[j].wait()

                @plsc.parallel_loop(j * 128, (j + 1) * 128, step=1, unroll=4)
                def _mbody(eidx):
                    ev16 = jnp.zeros((16,), I32) + eidx
                    for hd in range(NHEADS):
                        s = plsc.load_gather(ee_v, [ev16 + hd * ew])
                        for cc in range(2):
                            c0 = (hd * 2 + cc) * 16
                            rows_v[eidx, pl.ds(c0, 16)] = \
                                rows_v[eidx, pl.ds(c0, 16)] * s

                sdesc.append(
                    pltpu.async_copy(rows_v.at[pl.ds(j * 128, 128)],
                                     shout.at[didx_v.at[j]], ssem, add=True))

            for d in sdesc:
                d.wait()
            plsc.subcore_barrier()
            pltpu.sync_copy(shout.at[pl.ds(sid * n16, n16)],
                            convraw_hbm.at[g, cid, pl.ds(sid * n16, n16)])
            return 0
        lax.fori_loop(0, ng, gbody, 0)

    return gat


def _block_body(convraw_ref, dpart_ref, s128_ref, bias_ref, prow_ref,
                wq_ref, wk_ref, lng_ref, lnb_ref, xout_ref):
    raw = convraw_ref[0, 0] + convraw_ref[0, 1]
    dsum = dpart_ref[0, 0]
    for i in range(1, NWORK):
        dsum = dsum + dpart_ref[0, i]
    denom = lax.dot_general(dsum, s128_ref[...],
                            (((0,), (1,)), ((), ())),
                            preferred_element_type=F32)
    conv = raw / (denom + 1e-16) + bias_ref[0]
    h2 = jnp.dot(conv, prow_ref[0], preferred_element_type=F32)
    h2b = h2.astype(jnp.bfloat16)
    temp = jnp.zeros_like(h2)
    for hd in range(NHEADS):
        q = jnp.dot(h2, wq_ref[0, hd], preferred_element_type=F32)
        k = jnp.dot(h2, wk_ref[0, hd], preferred_element_type=F32)
        s = lax.dot_general(q, k, (((1,), (1,)), ((), ())),
                            preferred_element_type=F32) * (1.0 / jnp.sqrt(32.0))
        s = s - jnp.max(s, axis=-1, keepdims=True)
        p = jnp.exp(s)
        p = p / jnp.sum(p, axis=-1, keepdims=True)
        temp = temp + jnp.dot(p.astype(jnp.bfloat16), h2b,
                              preferred_element_type=F32)
    y = temp + h2
    m = jnp.mean(y, axis=-1, keepdims=True)
    v = jnp.mean((y - m) * (y - m), axis=-1, keepdims=True)
    xout_ref[0] = lng_ref[0] * (y - m) / jnp.sqrt(v + 1e-5) + lnb_ref[0]


def _block_tc(ng, n, convraw, dpart, s128, bias, prow, wq, wk, lng, lnb):
    return pl.pallas_call(
        _block_body,
        grid=(ng,),
        in_specs=[
            pl.BlockSpec((1, NCORE, n, DD), lambda g: (g, 0, 0, 0)),
            pl.BlockSpec((1, NWORK, NHEADS, n), lambda g: (g, 0, 0, 0)),
            pl.BlockSpec((DD, NHEADS), lambda g: (0, 0)),
            pl.BlockSpec((1, 1, DD), lambda g: (g, 0, 0)),
            pl.BlockSpec((1, DD, DD), lambda g: (g, 0, 0)),
            pl.BlockSpec((1, NHEADS, DD, NHID), lambda g: (g, 0, 0, 0)),
            pl.BlockSpec((1, NHEADS, DD, NHID), lambda g: (g, 0, 0, 0)),
            pl.BlockSpec((1, 1, DD), lambda g: (g, 0, 0)),
            pl.BlockSpec((1, 1, DD), lambda g: (g, 0, 0)),
        ],
        out_specs=pl.BlockSpec((1, n, DD), lambda g: (g, 0, 0)),
        out_shape=jax.ShapeDtypeStruct((ng, n, DD), F32),
    )(convraw, dpart, s128, bias, prow, wq, wk, lng, lnb)


def _make_pool_sc(ng, n):
    nw = n // NWORK
    n16 = n // NSUB
    mesh = plsc.VectorSubcoreMesh(core_axis_name="c", subcore_axis_name="s")

    @functools.partial(
        pl.kernel,
        mesh=mesh,
        compiler_params=pltpu.CompilerParams(needs_layout_passes=False),
        out_type=[
            jax.ShapeDtypeStruct((ng, NCORE, n, DD), F32),
            jax.ShapeDtypeStruct((ng, NWORK, n), F32),
        ],
        scratch_types=[
            pltpu.VMEM((nw, DD), F32),
            pltpu.VMEM((KNODE, nw), I32),
            pltpu.VMEM((n,), F32),
            pltpu.VMEM((n16, DD), F32),
            pltpu.VMEM_SHARED((n, DD), F32),
            pltpu.SemaphoreType.DMA,
        ],
    )
    def pool(x_hbm, node_hbm, pooled_hbm, cpart_hbm,
             rows_v, idx_v, cnt_v, zero_v, shpool, sem):
        cid = lax.axis_index("c")
        sid = lax.axis_index("s")
        wid = cid * NSUB + sid
        base = wid * nw
        zv16 = jnp.zeros((16,), F32)
        ones16 = jnp.full((16,), 1.0, F32)

        def zrow(r, _):
            for c in range(DD // 16):
                zero_v[r, pl.ds(c * 16, 16)] = zv16
            return 0
        lax.fori_loop(0, n16, zrow, 0)

        def gbody(g, _):
            pltpu.sync_copy(x_hbm.at[g, pl.ds(base, nw)], rows_v)
            for k in range(KNODE):
                pltpu.sync_copy(node_hbm.at[g, k, pl.ds(base, nw)], idx_v.at[k])
            pltpu.sync_copy(zero_v, shpool.at[pl.ds(sid * n16, n16)])

            def czero(i, _):
                cnt_v[pl.ds(i * 16, 16)] = zv16
                return 0
            lax.fori_loop(0, n // 16, czero, 0)

            for k in range(KNODE):
                for c in range(nw // 16):
                    idxv = idx_v[k, pl.ds(c * 16, 16)]
                    plsc.addupdate_scatter(cnt_v, [idxv], ones16)
            pltpu.sync_copy(cnt_v, cpart_hbm.at[g, wid])
            plsc.subcore_barrier()
            sdesc = [pltpu.async_copy(rows_v, shpool.at[idx_v.at[k]],
                                      sem, add=True)
                     for k in range(KNODE)]
            for d in sdesc:
                d.wait()
            plsc.subcore_barrier()
            pltpu.sync_copy(shpool.at[pl.ds(sid * n16, n16)],
                            pooled_hbm.at[g, cid, pl.ds(sid * n16, n16)])
            return 0
        lax.fori_loop(0, ng, gbody, 0)

    return pool


def _poolnorm_body(pooled_ref, cpart_ref, out_ref):
    pool = pooled_ref[0, 0] + pooled_ref[0, 1]
    cnt2d = cpart_ref[0]
    ones = jnp.ones((NWORK, 1), F32)
    cnt = lax.dot_general(cnt2d, ones, (((0,), (0,)), ((), ())),
                          preferred_element_type=F32)
    out_ref[...] = pool / jnp.maximum(cnt, 1.0)


def _poolnorm_tc(ng, n, pooled, cpart):
    return pl.pallas_call(
        _poolnorm_body,
        grid=(ng,),
        in_specs=[
            pl.BlockSpec((1, NCORE, n, DD), lambda g: (g, 0, 0, 0)),
            pl.BlockSpec((1, NWORK, n), lambda g: (g, 0, 0)),
        ],
        out_specs=pl.BlockSpec((n, DD), lambda g: (0, g)),
        out_shape=jax.ShapeDtypeStruct((n, ng * DD), F32),
    )(pooled, cpart)


def _make_pair_sc(npair, dcol):
    pw = npair // NWORK
    mesh = plsc.VectorSubcoreMesh(core_axis_name="c", subcore_axis_name="s")

    @functools.partial(
        pl.kernel,
        mesh=mesh,
        compiler_params=pltpu.CompilerParams(needs_layout_passes=False),
        out_type=[
            jax.ShapeDtypeStruct((npair, dcol), F32),
            jax.ShapeDtypeStruct((npair, dcol), F32),
        ],
        scratch_types=[
            pltpu.VMEM((2, pw), I32),
            pltpu.VMEM((pw, dcol), F32),
            pltpu.SemaphoreType.DMA,
        ],
    )
    def pair(oc_hbm, od_hbm, idxt_hbm, gc_hbm, gd_hbm, pidx_v, rows_v, sem):
        cid = lax.axis_index("c")
        sid = lax.axis_index("s")
        wid = cid * NSUB + sid
        base = wid * pw
        pltpu.sync_copy(idxt_hbm.at[0, pl.ds(base, pw)], pidx_v.at[0])
        pltpu.sync_copy(idxt_hbm.at[1, pl.ds(base, pw)], pidx_v.at[1])
        pltpu.async_copy(oc_hbm.at[pidx_v.at[0]], rows_v, sem).wait()
        pltpu.sync_copy(rows_v, gc_hbm.at[pl.ds(base, pw)])
        pltpu.async_copy(od_hbm.at[pidx_v.at[1]], rows_v, sem).wait()
        pltpu.sync_copy(rows_v, gd_hbm.at[pl.ds(base, pw)])

    return pair


def _mlp_body(gc_ref, gd_ref, w1c_ref, w1d_ref, b1_ref, w2_ref, b2_ref,
              w3_ref, b3_ref, out_ref):
    x = (jnp.dot(gc_ref[...], w1c_ref[...], preferred_element_type=F32)
         + jnp.dot(gd_ref[...], w1d_ref[...], preferred_element_type=F32)
         + b1_ref[...])
    x = jnp.maximum(x, 0.0)
    x = jnp.dot(x, w2_ref[...], preferred_element_type=F32) + b2_ref[...]
    x = jnp.maximum(x, 0.0)
    out_ref[...] = jnp.dot(x, w3_ref[...], preferred_element_type=F32) \
        + b3_ref[...]


def _mlp_tc(npair, dcol, gc, gd, w1c, w1d, b1, w2, b2, w3, b3):
    bp = 512
    return pl.pallas_call(
        _mlp_body,
        grid=(npair // bp,),
        in_specs=[
            pl.BlockSpec((bp, dcol), lambda g: (g, 0)),
            pl.BlockSpec((bp, dcol), lambda g: (g, 0)),
            pl.BlockSpec((dcol, 256), lambda g: (0, 0)),
            pl.BlockSpec((dcol, 256), lambda g: (0, 0)),
            pl.BlockSpec((1, 256), lambda g: (0, 0)),
            pl.BlockSpec((256, 256), lambda g: (0, 0)),
            pl.BlockSpec((1, 256), lambda g: (0, 0)),
            pl.BlockSpec((256, 1), lambda g: (0, 0)),
            pl.BlockSpec((1, 1), lambda g: (0, 0)),
        ],
        out_specs=pl.BlockSpec((bp, 1), lambda g: (g, 0)),
        out_shape=jax.ShapeDtypeStruct((npair, 1), F32),
    )(gc, gd, w1c, w1d, b1, w2, b2, w3, b3)


def _stack(params, prefix, name):
    return jnp.stack([params['%s_%d_%d' % (prefix, i, j)][name]
                      for i in range(2) for j in range(3)])


def _graph_tower(feat, edges, node_t, params, prefix, n, e, fin):
    ng = 6
    s8s = (jnp.arange(DD)[:, None] // NHID ==
           jnp.arange(8)[None, :]).astype(F32)
    s8d = (jnp.arange(DD)[:, None] // NHID ==
           (jnp.arange(8)[None, :] - 4)).astype(F32)
    s128 = (jnp.arange(DD)[:, None] // NHID ==
            jnp.arange(NHEADS)[None, :]).astype(F32)
    gat_sc = _make_gat_sc(ng, n, e)
    x = feat
    fin_l = fin
    for lname in (prefix, prefix + '2'):
        w = _stack(params, lname, 'W')
        asrc = _stack(params, lname, 'att_src').reshape(ng, 1, DD)
        adst = _stack(params, lname, 'att_dst').reshape(ng, 1, DD)
        h, atab = _feat_tc(ng, n, fin_l, x, w, asrc, adst, s8s, s8d)
        convraw, dpart = gat_sc(edges, h.reshape(ng * n, DD),
                                atab.reshape(ng, n * 8))
        dpart = dpart.reshape(ng, NWORK, NHEADS, n)
        x = _block_tc(ng, n, convraw, dpart, s128,
                      _stack(params, lname, 'gat_bias').reshape(ng, 1, DD),
                      _stack(params, lname, 'pro_W'),
                      _stack(params, lname, 'sa_Wq'),
                      _stack(params, lname, 'sa_Wk'),
                      _stack(params, lname, 'ln_g').reshape(ng, 1, DD),
                      _stack(params, lname, 'ln_b').reshape(ng, 1, DD))
        fin_l = DD
    pooled, cpart = _make_pool_sc(ng, n)(x, node_t)
    return _poolnorm_tc(ng, n, pooled, cpart)


def kernel(cell_adj_matrix, cell_feat_matrix, cell_node_set, drug_adj_matrix,
           drug_feat_matrix, drug_node_set, idx_cell_drug, params):
    ncell, fcell = cell_feat_matrix.shape[2], cell_feat_matrix.shape[3]
    ndrug, fdrug = drug_feat_matrix.shape[2], drug_feat_matrix.shape[3]
    ecell = cell_adj_matrix.shape[3]
    edrug = drug_adj_matrix.shape[3]
    npair = idx_cell_drug.shape[0]

    cell_edges = cell_adj_matrix.reshape(6, 2, ecell).astype(I32)
    drug_edges = drug_adj_matrix.reshape(6, 2, edrug).astype(I32)
    cell_node_t = jnp.transpose(
        cell_node_set.reshape(6, ncell, KNODE), (0, 2, 1)).astype(I32)
    drug_node_t = jnp.transpose(
        drug_node_set.reshape(6, ndrug, KNODE), (0, 2, 1)).astype(I32)

    oc = _graph_tower(cell_feat_matrix.reshape(6, ncell, fcell), cell_edges,
                      cell_node_t, params, 'cell', ncell, ecell, fcell)
    od = _graph_tower(drug_feat_matrix.reshape(6, ndrug, fdrug), drug_edges,
                      drug_node_t, params, 'drug', ndrug, edrug, fdrug)

    dcol = 6 * DD
    idxt = jnp.transpose(idx_cell_drug, (1, 0)).astype(I32)
    gc, gd = _make_pair_sc(npair, dcol)(oc, od, idxt)
    out = _mlp_tc(npair, dcol, gc, gd,
                  params['fc1_W'][:dcol], params['fc1_W'][dcol:],
                  params['fc1_b'].reshape(1, 256),
                  params['fc2_W'], params['fc2_b'].reshape(1, 256),
                  params['fc3_W'], params['fc3_b'].reshape(1, 1))
    return out.reshape(npair)


# speedup vs baseline: 1.1159x; 1.1159x over previous
import functools

import jax
import jax.numpy as jnp
from jax import lax
from jax.experimental import pallas as pl
from jax.experimental.pallas import tpu as pltpu
from jax.experimental.pallas import tpu_sc as plsc

F32 = jnp.float32
I32 = jnp.int32
NHEADS = 4
NHID = 32
DD = NHEADS * NHID
KNODE = 4
NCORE = 2
NSUB = 16
NWORK = NCORE * NSUB


def _feat_body(x_ref, w_ref, asrc_ref, adst_ref, s8s_ref, s8d_ref, h_ref, atab_ref):
    x = x_ref[0]
    w = w_ref[0]
    h = jnp.dot(x, w, preferred_element_type=F32)
    h_ref[0] = h
    asrc = asrc_ref[0]
    adst = adst_ref[0]
    atab_ref[0] = (lax.dot_general(s8s_ref[...], h * asrc,
                                   (((0,), (1,)), ((), ())),
                                   preferred_element_type=F32)
                   + lax.dot_general(s8d_ref[...], h * adst,
                                     (((0,), (1,)), ((), ())),
                                     preferred_element_type=F32))


def _feat_tc(ng, n, fin, x, w, asrc, adst, s8s, s8d):
    return pl.pallas_call(
        _feat_body,
        grid=(ng,),
        in_specs=[
            pl.BlockSpec((1, n, fin), lambda g: (g, 0, 0)),
            pl.BlockSpec((1, fin, DD), lambda g: (g, 0, 0)),
            pl.BlockSpec((1, 1, DD), lambda g: (g, 0, 0)),
            pl.BlockSpec((1, 1, DD), lambda g: (g, 0, 0)),
            pl.BlockSpec((DD, 8), lambda g: (0, 0)),
            pl.BlockSpec((DD, 8), lambda g: (0, 0)),
        ],
        out_specs=[
            pl.BlockSpec((1, n, DD), lambda g: (g, 0, 0)),
            pl.BlockSpec((1, 8, n), lambda g: (g, 0, 0)),
        ],
        out_shape=[
            jax.ShapeDtypeStruct((ng, n, DD), F32),
            jax.ShapeDtypeStruct((ng, 8, n), F32),
        ],
    )(x, w, asrc, adst, s8s, s8d)


def _make_gat_sc(ng, n, e):
    ew = e // NWORK
    nch = ew // 128
    nvr = ew // 16
    n16 = n // NSUB
    mesh = plsc.VectorSubcoreMesh(core_axis_name="c", subcore_axis_name="s")

    @functools.partial(
        pl.kernel,
        mesh=mesh,
        compiler_params=pltpu.CompilerParams(needs_layout_passes=False),
        out_type=[
            jax.ShapeDtypeStruct((ng, NCORE, n, DD), F32),
            jax.ShapeDtypeStruct((ng, NWORK * NHEADS, n), F32),
        ],
        scratch_types=[
            pltpu.VMEM((ew,), I32),
            pltpu.VMEM((ew,), I32),
            pltpu.VMEM((8, n), F32),
            pltpu.VMEM((NHEADS * ew,), F32),
            pltpu.VMEM((NHEADS, n), F32),
            pltpu.VMEM((nch, 128), I32),
            pltpu.VMEM((nch, 128), I32),
            pltpu.VMEM((ew, DD), F32),
            pltpu.VMEM((n16, DD), F32),
            pltpu.VMEM_SHARED((n, DD), F32),
            pltpu.SemaphoreType.DMA,
            pltpu.SemaphoreType.DMA,
        ],
    )
    def gat(edges_hbm, h_hbm, atab_hbm, convraw_hbm, dpart_hbm,
            src_v, dst_v, atab_v, ee_v, den_v, gidx_v, didx_v, rows_v,
            zero_v, shout, gsem, ssem):
        cid = lax.axis_index("c")
        sid = lax.axis_index("s")
        wid = cid * NSUB + sid
        base = wid * ew
        zv16 = jnp.zeros((16,), F32)

        def zrow(r, _):
            for c in range(DD // 16):
                zero_v[r, pl.ds(c * 16, 16)] = zv16
            return 0
        lax.fori_loop(0, n16, zrow, 0)

        def gbody(g, _):
            pltpu.sync_copy(edges_hbm.at[g, 0, pl.ds(base, ew)], src_v)
            pltpu.sync_copy(edges_hbm.at[g, 1, pl.ds(base, ew)], dst_v)

            @plsc.parallel_loop(0, nvr, step=1, unroll=4)
            def _ibody(i):
                srcv = src_v[pl.ds(i * 16, 16)]
                gidx_v[i // 8, pl.ds((i % 8) * 16, 16)] = srcv + g * n
                didx_v[i // 8, pl.ds((i % 8) * 16, 16)] = dst_v[pl.ds(i * 16, 16)]

            gdesc = [pltpu.async_copy(h_hbm.at[gidx_v.at[j]],
                                      rows_v.at[pl.ds(j * 128, 128)], gsem)
                     for j in range(nch)]

            pltpu.sync_copy(atab_hbm.at[g], atab_v)
            pltpu.sync_copy(zero_v, shout.at[pl.ds(sid * n16, n16)])

            @plsc.parallel_loop(0, n // 16, step=1, unroll=4)
            def _dzero(i):
                for hd in range(NHEADS):
                    den_v[hd, pl.ds(i * 16, 16)] = zv16

            def ebody(i, _):
                srcv = src_v[pl.ds(i * 16, 16)]
                dstv = dst_v[pl.ds(i * 16, 16)]
                for hd in range(NHEADS):
                    h16 = jnp.full((16,), hd, I32)
                    a_s = plsc.load_gather(atab_v, [h16, srcv])
                    a_d = plsc.load_gather(atab_v, [h16 + 4, dstv])
                    ev = a_s + a_d
                    ev = jnp.where(ev >= 0.0, ev, ev * 0.2)
                    eev = jnp.exp(ev)
                    ee_v[pl.ds(hd * ew + i * 16, 16)] = eev
                    plsc.addupdate_scatter(den_v, [h16, dstv], eev)
                return 0
            lax.fori_loop(0, nvr, ebody, 0)

            pltpu.sync_copy(den_v,
                            dpart_hbm.at[g, pl.ds(wid * NHEADS, NHEADS)])
            plsc.subcore_barrier()

            sdesc = []
            for j in range(nch):
                gdesc[j].wait()

                @plsc.parallel_loop(j * 128, (j + 1) * 128, step=1, unroll=4)
                def _mbody(eidx):
                    ev16 = jnp.zeros((16,), I32) + eidx
                    for hd in range(NHEADS):
                        s = plsc.load_gather(ee_v, [ev16 + hd * ew])
                        for cc in range(2):
                            c0 = (hd * 2 + cc) * 16
                            rows_v[eidx, pl.ds(c0, 16)] = \
                                rows_v[eidx, pl.ds(c0, 16)] * s

                sdesc.append(
                    pltpu.async_copy(rows_v.at[pl.ds(j * 128, 128)],
                                     shout.at[didx_v.at[j]], ssem, add=True))

            for d in sdesc:
                d.wait()
            plsc.subcore_barrier()
            pltpu.sync_copy(shout.at[pl.ds(sid * n16, n16)],
                            convraw_hbm.at[g, cid, pl.ds(sid * n16, n16)])
            return 0
        lax.fori_loop(0, ng, gbody, 0)

    return gat


def _block_body(convraw_ref, dpart_ref, s128_ref, bias_ref, prow_ref,
                wq_ref, wk_ref, lng_ref, lnb_ref, xout_ref):
    raw = convraw_ref[0, 0] + convraw_ref[0, 1]
    dsum = dpart_ref[0, pl.ds(0, 2 * NHEADS)]
    for i in range(1, NWORK // 2):
        dsum = dsum + dpart_ref[0, pl.ds(i * 2 * NHEADS, 2 * NHEADS)]
    denom = lax.dot_general(dsum, s128_ref[...],
                            (((0,), (1,)), ((), ())),
                            preferred_element_type=F32)
    conv = raw / (denom + 1e-16) + bias_ref[0]
    h2 = jnp.dot(conv, prow_ref[0], preferred_element_type=F32)
    temp = jnp.zeros_like(h2)
    for hd in range(NHEADS):
        q = jnp.dot(h2, wq_ref[0, hd], preferred_element_type=F32)
        k = jnp.dot(h2, wk_ref[0, hd], preferred_element_type=F32)
        s = lax.dot_general(q, k, (((1,), (1,)), ((), ())),
                            preferred_element_type=F32) * (1.0 / jnp.sqrt(32.0))
        s = s - jnp.max(s, axis=-1, keepdims=True)
        p = jnp.exp(s)
        p = p / jnp.sum(p, axis=-1, keepdims=True)
        temp = temp + jnp.dot(p, h2, preferred_element_type=F32)
    y = temp + h2
    m = jnp.mean(y, axis=-1, keepdims=True)
    v = jnp.mean((y - m) * (y - m), axis=-1, keepdims=True)
    xout_ref[0] = lng_ref[0] * (y - m) / jnp.sqrt(v + 1e-5) + lnb_ref[0]


def _block_tc(ng, n, convraw, dpart, s128, bias, prow, wq, wk, lng, lnb):
    return pl.pallas_call(
        _block_body,
        grid=(ng,),
        in_specs=[
            pl.BlockSpec((1, NCORE, n, DD), lambda g: (g, 0, 0, 0)),
            pl.BlockSpec((1, NWORK * NHEADS, n), lambda g: (g, 0, 0)),
            pl.BlockSpec((DD, 8), lambda g: (0, 0)),
            pl.BlockSpec((1, 1, DD), lambda g: (g, 0, 0)),
            pl.BlockSpec((1, DD, DD), lambda g: (g, 0, 0)),
            pl.BlockSpec((1, NHEADS, DD, NHID), lambda g: (g, 0, 0, 0)),
            pl.BlockSpec((1, NHEADS, DD, NHID), lambda g: (g, 0, 0, 0)),
            pl.BlockSpec((1, 1, DD), lambda g: (g, 0, 0)),
            pl.BlockSpec((1, 1, DD), lambda g: (g, 0, 0)),
        ],
        out_specs=pl.BlockSpec((1, n, DD), lambda g: (g, 0, 0)),
        out_shape=jax.ShapeDtypeStruct((ng, n, DD), F32),
    )(convraw, dpart, s128, bias, prow, wq, wk, lng, lnb)


def _make_pool_sc(ng, n):
    nw = n // NWORK
    n16 = n // NSUB
    mesh = plsc.VectorSubcoreMesh(core_axis_name="c", subcore_axis_name="s")

    @functools.partial(
        pl.kernel,
        mesh=mesh,
        compiler_params=pltpu.CompilerParams(needs_layout_passes=False),
        out_type=[
            jax.ShapeDtypeStruct((ng, NCORE, n, DD), F32),
            jax.ShapeDtypeStruct((ng, NWORK, n), F32),
        ],
        scratch_types=[
            pltpu.VMEM((nw, DD), F32),
            pltpu.VMEM((KNODE, nw), I32),
            pltpu.VMEM((n,), F32),
            pltpu.VMEM((n16, DD), F32),
            pltpu.VMEM_SHARED((n, DD), F32),
            pltpu.SemaphoreType.DMA,
        ],
    )
    def pool(x_hbm, node_hbm, pooled_hbm, cpart_hbm,
             rows_v, idx_v, cnt_v, zero_v, shpool, sem):
        cid = lax.axis_index("c")
        sid = lax.axis_index("s")
        wid = cid * NSUB + sid
        base = wid * nw
        zv16 = jnp.zeros((16,), F32)
        ones16 = jnp.full((16,), 1.0, F32)

        def zrow(r, _):
            for c in range(DD // 16):
                zero_v[r, pl.ds(c * 16, 16)] = zv16
            return 0
        lax.fori_loop(0, n16, zrow, 0)

        def gbody(g, _):
            pltpu.sync_copy(x_hbm.at[g, pl.ds(base, nw)], rows_v)
            for k in range(KNODE):
                pltpu.sync_copy(node_hbm.at[g, k, pl.ds(base, nw)], idx_v.at[k])
            pltpu.sync_copy(zero_v, shpool.at[pl.ds(sid * n16, n16)])

            def czero(i, _):
                cnt_v[pl.ds(i * 16, 16)] = zv16
                return 0
            lax.fori_loop(0, n // 16, czero, 0)

            for k in range(KNODE):
                for c in range(nw // 16):
                    idxv = idx_v[k, pl.ds(c * 16, 16)]
                    plsc.addupdate_scatter(cnt_v, [idxv], ones16)
            pltpu.sync_copy(cnt_v, cpart_hbm.at[g, wid])
            plsc.subcore_barrier()
            sdesc = [pltpu.async_copy(rows_v, shpool.at[idx_v.at[k]],
                                      sem, add=True)
                     for k in range(KNODE)]
            for d in sdesc:
                d.wait()
            plsc.subcore_barrier()
            pltpu.sync_copy(shpool.at[pl.ds(sid * n16, n16)],
                            pooled_hbm.at[g, cid, pl.ds(sid * n16, n16)])
            return 0
        lax.fori_loop(0, ng, gbody, 0)

    return pool


def _poolnorm_body(pooled_ref, cpart_ref, out_ref):
    pool = pooled_ref[0, 0] + pooled_ref[0, 1]
    cnt2d = cpart_ref[0]
    ones = jnp.ones((NWORK, 1), F32)
    cnt = lax.dot_general(cnt2d, ones, (((0,), (0,)), ((), ())),
                          preferred_element_type=F32)
    out_ref[...] = pool / jnp.maximum(cnt, 1.0)


def _poolnorm_tc(ng, n, pooled, cpart):
    return pl.pallas_call(
        _poolnorm_body,
        grid=(ng,),
        in_specs=[
            pl.BlockSpec((1, NCORE, n, DD), lambda g: (g, 0, 0, 0)),
            pl.BlockSpec((1, NWORK, n), lambda g: (g, 0, 0)),
        ],
        out_specs=pl.BlockSpec((n, DD), lambda g: (0, g)),
        out_shape=jax.ShapeDtypeStruct((n, ng * DD), F32),
    )(pooled, cpart)


def _make_pair_sc(npair, dcol):
    pw = npair // NWORK
    mesh = plsc.VectorSubcoreMesh(core_axis_name="c", subcore_axis_name="s")

    @functools.partial(
        pl.kernel,
        mesh=mesh,
        compiler_params=pltpu.CompilerParams(needs_layout_passes=False),
        out_type=[
            jax.ShapeDtypeStruct((npair, dcol), F32),
            jax.ShapeDtypeStruct((npair, dcol), F32),
        ],
        scratch_types=[
            pltpu.VMEM((2, pw), I32),
            pltpu.VMEM((pw, dcol), F32),
            pltpu.SemaphoreType.DMA,
        ],
    )
    def pair(oc_hbm, od_hbm, idxt_hbm, gc_hbm, gd_hbm, pidx_v, rows_v, sem):
        cid = lax.axis_index("c")
        sid = lax.axis_index("s")
        wid = cid * NSUB + sid
        base = wid * pw
        pltpu.sync_copy(idxt_hbm.at[0, pl.ds(base, pw)], pidx_v.at[0])
        pltpu.sync_copy(idxt_hbm.at[1, pl.ds(base, pw)], pidx_v.at[1])
        pltpu.async_copy(oc_hbm.at[pidx_v.at[0]], rows_v, sem).wait()
        pltpu.sync_copy(rows_v, gc_hbm.at[pl.ds(base, pw)])
        pltpu.async_copy(od_hbm.at[pidx_v.at[1]], rows_v, sem).wait()
        pltpu.sync_copy(rows_v, gd_hbm.at[pl.ds(base, pw)])

    return pair


def _mlp_body(gc_ref, gd_ref, w1c_ref, w1d_ref, b1_ref, w2_ref, b2_ref,
              w3_ref, b3_ref, out_ref):
    x = (jnp.dot(gc_ref[...], w1c_ref[...], preferred_element_type=F32)
         + jnp.dot(gd_ref[...], w1d_ref[...], preferred_element_type=F32)
         + b1_ref[...])
    x = jnp.maximum(x, 0.0)
    x = jnp.dot(x, w2_ref[...], preferred_element_type=F32) + b2_ref[...]
    x = jnp.maximum(x, 0.0)
    out_ref[...] = jnp.dot(x, w3_ref[...], preferred_element_type=F32) \
        + b3_ref[...]


def _mlp_tc(npair, dcol, gc, gd, w1c, w1d, b1, w2, b2, w3, b3):
    bp = 512
    return pl.pallas_call(
        _mlp_body,
        grid=(npair // bp,),
        in_specs=[
            pl.BlockSpec((bp, dcol), lambda g: (g, 0)),
            pl.BlockSpec((bp, dcol), lambda g: (g, 0)),
            pl.BlockSpec((dcol, 256), lambda g: (0, 0)),
            pl.BlockSpec((dcol, 256), lambda g: (0, 0)),
            pl.BlockSpec((1, 256), lambda g: (0, 0)),
            pl.BlockSpec((256, 256), lambda g: (0, 0)),
            pl.BlockSpec((1, 256), lambda g: (0, 0)),
            pl.BlockSpec((256, 1), lambda g: (0, 0)),
            pl.BlockSpec((1, 1), lambda g: (0, 0)),
        ],
        out_specs=pl.BlockSpec((bp, 1), lambda g: (g, 0)),
        out_shape=jax.ShapeDtypeStruct((npair, 1), F32),
    )(gc, gd, w1c, w1d, b1, w2, b2, w3, b3)


def _stack(params, prefix, name):
    return jnp.stack([params['%s_%d_%d' % (prefix, i, j)][name]
                      for i in range(2) for j in range(3)])


def _graph_tower(feat, edges, node_t, params, prefix, n, e, fin):
    ng = 6
    s8s = (jnp.arange(DD)[:, None] // NHID ==
           jnp.arange(8)[None, :]).astype(F32)
    s8d = (jnp.arange(DD)[:, None] // NHID ==
           (jnp.arange(8)[None, :] - 4)).astype(F32)
    s128 = s8s + s8d
    gat_sc = _make_gat_sc(ng, n, e)
    x = feat
    fin_l = fin
    for lname in (prefix, prefix + '2'):
        w = _stack(params, lname, 'W')
        asrc = _stack(params, lname, 'att_src').reshape(ng, 1, DD)
        adst = _stack(params, lname, 'att_dst').reshape(ng, 1, DD)
        h, atab = _feat_tc(ng, n, fin_l, x, w, asrc, adst, s8s, s8d)
        convraw, dpart = gat_sc(edges, h.reshape(ng * n, DD), atab)
        x = _block_tc(ng, n, convraw, dpart, s128,
                      _stack(params, lname, 'gat_bias').reshape(ng, 1, DD),
                      _stack(params, lname, 'pro_W'),
                      _stack(params, lname, 'sa_Wq'),
                      _stack(params, lname, 'sa_Wk'),
                      _stack(params, lname, 'ln_g').reshape(ng, 1, DD),
                      _stack(params, lname, 'ln_b').reshape(ng, 1, DD))
        fin_l = DD
    pooled, cpart = _make_pool_sc(ng, n)(x, node_t)
    return _poolnorm_tc(ng, n, pooled, cpart)


def kernel(cell_adj_matrix, cell_feat_matrix, cell_node_set, drug_adj_matrix,
           drug_feat_matrix, drug_node_set, idx_cell_drug, params):
    ncell, fcell = cell_feat_matrix.shape[2], cell_feat_matrix.shape[3]
    ndrug, fdrug = drug_feat_matrix.shape[2], drug_feat_matrix.shape[3]
    ecell = cell_adj_matrix.shape[3]
    edrug = drug_adj_matrix.shape[3]
    npair = idx_cell_drug.shape[0]

    cell_edges = cell_adj_matrix.reshape(6, 2, ecell).astype(I32)
    drug_edges = drug_adj_matrix.reshape(6, 2, edrug).astype(I32)
    cell_node_t = jnp.transpose(
        cell_node_set.reshape(6, ncell, KNODE), (0, 2, 1)).astype(I32)
    drug_node_t = jnp.transpose(
        drug_node_set.reshape(6, ndrug, KNODE), (0, 2, 1)).astype(I32)

    oc = _graph_tower(cell_feat_matrix.reshape(6, ncell, fcell), cell_edges,
                      cell_node_t, params, 'cell', ncell, ecell, fcell)
    od = _graph_tower(drug_feat_matrix.reshape(6, ndrug, fdrug), drug_edges,
                      drug_node_t, params, 'drug', ndrug, edrug, fdrug)

    dcol = 6 * DD
    idxt = jnp.transpose(idx_cell_drug, (1, 0)).astype(I32)
    gc, gd = _make_pair_sc(npair, dcol)(oc, od, idxt)
    out = _mlp_tc(npair, dcol, gc, gd,
                  params['fc1_W'][:dcol], params['fc1_W'][dcol:],
                  params['fc1_b'].reshape(1, 256),
                  params['fc2_W'], params['fc2_b'].reshape(1, 256),
                  params['fc3_W'], params['fc3_b'].reshape(1, 1))
    return out.reshape(npair)


# speedup vs baseline: 1.1370x; 1.0189x over previous
import functools

import jax
import jax.numpy as jnp
from jax import lax
from jax.experimental import pallas as pl
from jax.experimental.pallas import tpu as pltpu
from jax.experimental.pallas import tpu_sc as plsc

F32 = jnp.float32
I32 = jnp.int32
NHEADS = 4
NHID = 32
DD = NHEADS * NHID
KNODE = 4
NCORE = 2
NSUB = 16
NWORK = NCORE * NSUB


def _feat_body(x_ref, w_ref, asrc_ref, adst_ref, s8s_ref, s8d_ref, h_ref, atab_ref):
    x = x_ref[0]
    w = w_ref[0]
    h = jnp.dot(x, w, preferred_element_type=F32)
    h_ref[0] = h
    asrc = asrc_ref[0]
    adst = adst_ref[0]
    atab_ref[0] = (lax.dot_general(s8s_ref[...], h * asrc,
                                   (((0,), (1,)), ((), ())),
                                   preferred_element_type=F32)
                   + lax.dot_general(s8d_ref[...], h * adst,
                                     (((0,), (1,)), ((), ())),
                                     preferred_element_type=F32))


def _feat_tc(ng, n, fin, x, w, asrc, adst, s8s, s8d):
    return pl.pallas_call(
        _feat_body,
        grid=(ng,),
        in_specs=[
            pl.BlockSpec((1, n, fin), lambda g: (g, 0, 0)),
            pl.BlockSpec((1, fin, DD), lambda g: (g, 0, 0)),
            pl.BlockSpec((1, 1, DD), lambda g: (g, 0, 0)),
            pl.BlockSpec((1, 1, DD), lambda g: (g, 0, 0)),
            pl.BlockSpec((DD, 8), lambda g: (0, 0)),
            pl.BlockSpec((DD, 8), lambda g: (0, 0)),
        ],
        out_specs=[
            pl.BlockSpec((1, n, DD), lambda g: (g, 0, 0)),
            pl.BlockSpec((1, 8, n), lambda g: (g, 0, 0)),
        ],
        out_shape=[
            jax.ShapeDtypeStruct((ng, n, DD), F32),
            jax.ShapeDtypeStruct((ng, 8, n), F32),
        ],
    )(x, w, asrc, adst, s8s, s8d)


def _make_gat_sc(ng, n, e):
    ew = e // NWORK
    nch = ew // 128
    nvr = ew // 16
    n16 = n // NSUB
    mesh = plsc.VectorSubcoreMesh(core_axis_name="c", subcore_axis_name="s")

    @functools.partial(
        pl.kernel,
        mesh=mesh,
        compiler_params=pltpu.CompilerParams(needs_layout_passes=False),
        out_type=[
            jax.ShapeDtypeStruct((ng, NCORE, n, DD), F32),
            jax.ShapeDtypeStruct((ng, NWORK * NHEADS, n), F32),
        ],
        scratch_types=[
            pltpu.VMEM((ew,), I32),
            pltpu.VMEM((ew,), I32),
            pltpu.VMEM((8, n), F32),
            pltpu.VMEM((NHEADS * ew,), F32),
            pltpu.VMEM((NHEADS, n), F32),
            pltpu.VMEM((nch, 128), I32),
            pltpu.VMEM((nch, 128), I32),
            pltpu.VMEM((ew, DD), F32),
            pltpu.VMEM((n16, DD), F32),
            pltpu.VMEM_SHARED((n, DD), F32),
            pltpu.SemaphoreType.DMA,
            pltpu.SemaphoreType.DMA,
        ],
    )
    def gat(edges_hbm, h_hbm, atab_hbm, convraw_hbm, dpart_hbm,
            src_v, dst_v, atab_v, ee_v, den_v, gidx_v, didx_v, rows_v,
            zero_v, shout, gsem, ssem):
        cid = lax.axis_index("c")
        sid = lax.axis_index("s")
        wid = cid * NSUB + sid
        base = wid * ew
        zv16 = jnp.zeros((16,), F32)

        def zrow(r, _):
            for c in range(DD // 16):
                zero_v[r, pl.ds(c * 16, 16)] = zv16
            return 0
        lax.fori_loop(0, n16, zrow, 0)

        def gbody(g, _):
            pltpu.sync_copy(edges_hbm.at[g, 0, pl.ds(base, ew)], src_v)
            pltpu.sync_copy(edges_hbm.at[g, 1, pl.ds(base, ew)], dst_v)

            @plsc.parallel_loop(0, nvr, step=1, unroll=4)
            def _ibody(i):
                srcv = src_v[pl.ds(i * 16, 16)]
                gidx_v[i // 8, pl.ds((i % 8) * 16, 16)] = srcv + g * n
                didx_v[i // 8, pl.ds((i % 8) * 16, 16)] = dst_v[pl.ds(i * 16, 16)]

            gdesc = [pltpu.async_copy(h_hbm.at[gidx_v.at[j]],
                                      rows_v.at[pl.ds(j * 128, 128)], gsem)
                     for j in range(nch)]

            pltpu.sync_copy(atab_hbm.at[g], atab_v)
            pltpu.sync_copy(zero_v, shout.at[pl.ds(sid * n16, n16)])

            @plsc.parallel_loop(0, n // 16, step=1, unroll=4)
            def _dzero(i):
                for hd in range(NHEADS):
                    den_v[hd, pl.ds(i * 16, 16)] = zv16

            def ebody(i, _):
                srcv = src_v[pl.ds(i * 16, 16)]
                dstv = dst_v[pl.ds(i * 16, 16)]
                for hd in range(NHEADS):
                    h16 = jnp.full((16,), hd, I32)
                    a_s = plsc.load_gather(atab_v, [h16, srcv])
                    a_d = plsc.load_gather(atab_v, [h16 + 4, dstv])
                    ev = a_s + a_d
                    ev = jnp.where(ev >= 0.0, ev, ev * 0.2)
                    eev = jnp.exp(ev)
                    ee_v[pl.ds(hd * ew + i * 16, 16)] = eev
                    plsc.addupdate_scatter(den_v, [h16, dstv], eev)
                return 0
            lax.fori_loop(0, nvr, ebody, 0)

            pltpu.sync_copy(den_v,
                            dpart_hbm.at[g, pl.ds(wid * NHEADS, NHEADS)])
            plsc.subcore_barrier()

            sdesc = []
            for j in range(nch):
                gdesc[j].wait()

                @plsc.parallel_loop(j * 128, (j + 1) * 128, step=1, unroll=4)
                def _mbody(eidx):
                    ev16 = jnp.zeros((16,), I32) + eidx
                    for hd in range(NHEADS):
                        s = plsc.load_gather(ee_v, [ev16 + hd * ew])
                        for cc in range(2):
                            c0 = (hd * 2 + cc) * 16
                            rows_v[eidx, pl.ds(c0, 16)] = \
                                rows_v[eidx, pl.ds(c0, 16)] * s

                sdesc.append(
                    pltpu.async_copy(rows_v.at[pl.ds(j * 128, 128)],
                                     shout.at[didx_v.at[j]], ssem, add=True))

            for d in sdesc:
                d.wait()
            plsc.subcore_barrier()
            pltpu.sync_copy(shout.at[pl.ds(sid * n16, n16)],
                            convraw_hbm.at[g, cid, pl.ds(sid * n16, n16)])
            return 0
        lax.fori_loop(0, ng, gbody, 0)

    return gat


def _block_core(convraw_ref, dpart_ref, s128_ref, bias_ref, prow_ref,
                wq_ref, wk_ref, lng_ref, lnb_ref):
    raw = convraw_ref[0, 0] + convraw_ref[0, 1]
    dsum = dpart_ref[0, pl.ds(0, 2 * NHEADS)]
    for i in range(1, NWORK // 2):
        dsum = dsum + dpart_ref[0, pl.ds(i * 2 * NHEADS, 2 * NHEADS)]
    denom = lax.dot_general(dsum, s128_ref[...],
                            (((0,), (1,)), ((), ())),
                            preferred_element_type=F32)
    conv = raw / (denom + 1e-16) + bias_ref[0]
    h2 = jnp.dot(conv, prow_ref[0], preferred_element_type=F32)
    temp = jnp.zeros_like(h2)
    for hd in range(NHEADS):
        q = jnp.dot(h2, wq_ref[0, hd], preferred_element_type=F32)
        k = jnp.dot(h2, wk_ref[0, hd], preferred_element_type=F32)
        s = lax.dot_general(q, k, (((1,), (1,)), ((), ())),
                            preferred_element_type=F32) * (1.0 / jnp.sqrt(32.0))
        p = jnp.exp(s)
        r = jnp.dot(p, h2, preferred_element_type=F32)
        temp = temp + r / jnp.sum(p, axis=-1, keepdims=True)
    y = temp + h2
    m = jnp.mean(y, axis=-1, keepdims=True)
    v = jnp.mean((y - m) * (y - m), axis=-1, keepdims=True)
    return lng_ref[0] * (y - m) / jnp.sqrt(v + 1e-5) + lnb_ref[0]


def _block_body(convraw_ref, dpart_ref, s128_ref, bias_ref, prow_ref,
                wq_ref, wk_ref, lng_ref, lnb_ref, xout_ref):
    xout_ref[0] = _block_core(convraw_ref, dpart_ref, s128_ref, bias_ref,
                              prow_ref, wq_ref, wk_ref, lng_ref, lnb_ref)


def _block_feat_body(convraw_ref, dpart_ref, s128_ref, bias_ref, prow_ref,
                     wq_ref, wk_ref, lng_ref, lnb_ref, w2_ref, asrc2_ref,
                     adst2_ref, s8s_ref, s8d_ref, h_ref, atab_ref):
    x = _block_core(convraw_ref, dpart_ref, s128_ref, bias_ref,
                    prow_ref, wq_ref, wk_ref, lng_ref, lnb_ref)
    h = jnp.dot(x, w2_ref[0], preferred_element_type=F32)
    h_ref[0] = h
    atab_ref[0] = (lax.dot_general(s8s_ref[...], h * asrc2_ref[0],
                                   (((0,), (1,)), ((), ())),
                                   preferred_element_type=F32)
                   + lax.dot_general(s8d_ref[...], h * adst2_ref[0],
                                     (((0,), (1,)), ((), ())),
                                     preferred_element_type=F32))


def _block_in_specs(n):
    return [
        pl.BlockSpec((1, NCORE, n, DD), lambda g: (g, 0, 0, 0)),
        pl.BlockSpec((1, NWORK * NHEADS, n), lambda g: (g, 0, 0)),
        pl.BlockSpec((DD, 8), lambda g: (0, 0)),
        pl.BlockSpec((1, 1, DD), lambda g: (g, 0, 0)),
        pl.BlockSpec((1, DD, DD), lambda g: (g, 0, 0)),
        pl.BlockSpec((1, NHEADS, DD, NHID), lambda g: (g, 0, 0, 0)),
        pl.BlockSpec((1, NHEADS, DD, NHID), lambda g: (g, 0, 0, 0)),
        pl.BlockSpec((1, 1, DD), lambda g: (g, 0, 0)),
        pl.BlockSpec((1, 1, DD), lambda g: (g, 0, 0)),
    ]


def _block_tc(ng, n, convraw, dpart, s128, bias, prow, wq, wk, lng, lnb):
    return pl.pallas_call(
        _block_body,
        grid=(ng,),
        in_specs=_block_in_specs(n),
        out_specs=pl.BlockSpec((1, n, DD), lambda g: (g, 0, 0)),
        out_shape=jax.ShapeDtypeStruct((ng, n, DD), F32),
    )(convraw, dpart, s128, bias, prow, wq, wk, lng, lnb)


def _block_feat_tc(ng, n, convraw, dpart, s128, bias, prow, wq, wk, lng, lnb,
                   w2, asrc2, adst2, s8s, s8d):
    return pl.pallas_call(
        _block_feat_body,
        grid=(ng,),
        in_specs=_block_in_specs(n) + [
            pl.BlockSpec((1, DD, DD), lambda g: (g, 0, 0)),
            pl.BlockSpec((1, 1, DD), lambda g: (g, 0, 0)),
            pl.BlockSpec((1, 1, DD), lambda g: (g, 0, 0)),
            pl.BlockSpec((DD, 8), lambda g: (0, 0)),
            pl.BlockSpec((DD, 8), lambda g: (0, 0)),
        ],
        out_specs=[
            pl.BlockSpec((1, n, DD), lambda g: (g, 0, 0)),
            pl.BlockSpec((1, 8, n), lambda g: (g, 0, 0)),
        ],
        out_shape=[
            jax.ShapeDtypeStruct((ng, n, DD), F32),
            jax.ShapeDtypeStruct((ng, 8, n), F32),
        ],
    )(convraw, dpart, s128, bias, prow, wq, wk, lng, lnb,
      w2, asrc2, adst2, s8s, s8d)


def _make_pool_sc(ng, n):
    nw = n // NWORK
    n16 = n // NSUB
    mesh = plsc.VectorSubcoreMesh(core_axis_name="c", subcore_axis_name="s")

    @functools.partial(
        pl.kernel,
        mesh=mesh,
        compiler_params=pltpu.CompilerParams(needs_layout_passes=False),
        out_type=[
            jax.ShapeDtypeStruct((ng, NCORE, n, DD), F32),
            jax.ShapeDtypeStruct((ng, NWORK, n), F32),
        ],
        scratch_types=[
            pltpu.VMEM((nw, DD), F32),
            pltpu.VMEM((KNODE, nw), I32),
            pltpu.VMEM((n,), F32),
            pltpu.VMEM((n16, DD), F32),
            pltpu.VMEM_SHARED((n, DD), F32),
            pltpu.SemaphoreType.DMA,
        ],
    )
    def pool(x_hbm, node_hbm, pooled_hbm, cpart_hbm,
             rows_v, idx_v, cnt_v, zero_v, shpool, sem):
        cid = lax.axis_index("c")
        sid = lax.axis_index("s")
        wid = cid * NSUB + sid
        base = wid * nw
        zv16 = jnp.zeros((16,), F32)
        ones16 = jnp.full((16,), 1.0, F32)

        def zrow(r, _):
            for c in range(DD // 16):
                zero_v[r, pl.ds(c * 16, 16)] = zv16
            return 0
        lax.fori_loop(0, n16, zrow, 0)

        def gbody(g, _):
            pltpu.sync_copy(x_hbm.at[g, pl.ds(base, nw)], rows_v)
            for k in range(KNODE):
                pltpu.sync_copy(node_hbm.at[g, k, pl.ds(base, nw)], idx_v.at[k])
            pltpu.sync_copy(zero_v, shpool.at[pl.ds(sid * n16, n16)])

            def czero(i, _):
                cnt_v[pl.ds(i * 16, 16)] = zv16
                return 0
            lax.fori_loop(0, n // 16, czero, 0)

            for k in range(KNODE):
                for c in range(nw // 16):
                    idxv = idx_v[k, pl.ds(c * 16, 16)]
                    plsc.addupdate_scatter(cnt_v, [idxv], ones16)
            pltpu.sync_copy(cnt_v, cpart_hbm.at[g, wid])
            plsc.subcore_barrier()
            sdesc = [pltpu.async_copy(rows_v, shpool.at[idx_v.at[k]],
                                      sem, add=True)
                     for k in range(KNODE)]
            for d in sdesc:
                d.wait()
            plsc.subcore_barrier()
            pltpu.sync_copy(shpool.at[pl.ds(sid * n16, n16)],
                            pooled_hbm.at[g, cid, pl.ds(sid * n16, n16)])
            return 0
        lax.fori_loop(0, ng, gbody, 0)

    return pool


def _poolnorm_body(pooled_ref, cpart_ref, out_ref):
    pool = pooled_ref[0, 0] + pooled_ref[0, 1]
    cnt2d = cpart_ref[0]
    ones = jnp.ones((NWORK, 1), F32)
    cnt = lax.dot_general(cnt2d, ones, (((0,), (0,)), ((), ())),
                          preferred_element_type=F32)
    out_ref[...] = pool / jnp.maximum(cnt, 1.0)


def _poolnorm_tc(ng, n, pooled, cpart):
    return pl.pallas_call(
        _poolnorm_body,
        grid=(ng,),
        in_specs=[
            pl.BlockSpec((1, NCORE, n, DD), lambda g: (g, 0, 0, 0)),
            pl.BlockSpec((1, NWORK, n), lambda g: (g, 0, 0)),
        ],
        out_specs=pl.BlockSpec((n, DD), lambda g: (0, g)),
        out_shape=jax.ShapeDtypeStruct((n, ng * DD), F32),
    )(pooled, cpart)


def _make_pair_sc(npair, dcol):
    pw = npair // NWORK
    mesh = plsc.VectorSubcoreMesh(core_axis_name="c", subcore_axis_name="s")

    @functools.partial(
        pl.kernel,
        mesh=mesh,
        compiler_params=pltpu.CompilerParams(needs_layout_passes=False),
        out_type=[
            jax.ShapeDtypeStruct((npair, dcol), F32),
            jax.ShapeDtypeStruct((npair, dcol), F32),
        ],
        scratch_types=[
            pltpu.VMEM((2, pw), I32),
            pltpu.VMEM((pw, dcol), F32),
            pltpu.SemaphoreType.DMA,
        ],
    )
    def pair(oc_hbm, od_hbm, idxt_hbm, gc_hbm, gd_hbm, pidx_v, rows_v, sem):
        cid = lax.axis_index("c")
        sid = lax.axis_index("s")
        wid = cid * NSUB + sid
        base = wid * pw
        pltpu.sync_copy(idxt_hbm.at[0, pl.ds(base, pw)], pidx_v.at[0])
        pltpu.sync_copy(idxt_hbm.at[1, pl.ds(base, pw)], pidx_v.at[1])
        pltpu.async_copy(oc_hbm.at[pidx_v.at[0]], rows_v, sem).wait()
        pltpu.sync_copy(rows_v, gc_hbm.at[pl.ds(base, pw)])
        pltpu.async_copy(od_hbm.at[pidx_v.at[1]], rows_v, sem).wait()
        pltpu.sync_copy(rows_v, gd_hbm.at[pl.ds(base, pw)])

    return pair


def _mlp_body(gc_ref, gd_ref, w1c_ref, w1d_ref, b1_ref, w2_ref, b2_ref,
              w3_ref, b3_ref, out_ref):
    x = (jnp.dot(gc_ref[...], w1c_ref[...], preferred_element_type=F32)
         + jnp.dot(gd_ref[...], w1d_ref[...], preferred_element_type=F32)
         + b1_ref[...])
    x = jnp.maximum(x, 0.0)
    x = jnp.dot(x, w2_ref[...], preferred_element_type=F32) + b2_ref[...]
    x = jnp.maximum(x, 0.0)
    out_ref[...] = jnp.dot(x, w3_ref[...], preferred_element_type=F32) \
        + b3_ref[...]


def _mlp_tc(npair, dcol, gc, gd, w1c, w1d, b1, w2, b2, w3, b3):
    bp = 512
    return pl.pallas_call(
        _mlp_body,
        grid=(npair // bp,),
        in_specs=[
            pl.BlockSpec((bp, dcol), lambda g: (g, 0)),
            pl.BlockSpec((bp, dcol), lambda g: (g, 0)),
            pl.BlockSpec((dcol, 256), lambda g: (0, 0)),
            pl.BlockSpec((dcol, 256), lambda g: (0, 0)),
            pl.BlockSpec((1, 256), lambda g: (0, 0)),
            pl.BlockSpec((256, 256), lambda g: (0, 0)),
            pl.BlockSpec((1, 256), lambda g: (0, 0)),
            pl.BlockSpec((256, 1), lambda g: (0, 0)),
            pl.BlockSpec((1, 1), lambda g: (0, 0)),
        ],
        out_specs=pl.BlockSpec((bp, 1), lambda g: (g, 0)),
        out_shape=jax.ShapeDtypeStruct((npair, 1), F32),
    )(gc, gd, w1c, w1d, b1, w2, b2, w3, b3)


def _stack(params, prefix, name):
    return jnp.stack([params['%s_%d_%d' % (prefix, i, j)][name]
                      for i in range(2) for j in range(3)])


def _graph_tower(feat, edges, node_t, params, prefix, n, e, fin):
    ng = 6
    s8s = (jnp.arange(DD)[:, None] // NHID ==
           jnp.arange(8)[None, :]).astype(F32)
    s8d = (jnp.arange(DD)[:, None] // NHID ==
           (jnp.arange(8)[None, :] - 4)).astype(F32)
    s128 = s8s + s8d
    gat_sc = _make_gat_sc(ng, n, e)
    l1, l2 = prefix, prefix + '2'
    h, atab = _feat_tc(
        ng, n, fin, feat, _stack(params, l1, 'W'),
        _stack(params, l1, 'att_src').reshape(ng, 1, DD),
        _stack(params, l1, 'att_dst').reshape(ng, 1, DD), s8s, s8d)
    convraw, dpart = gat_sc(edges, h.reshape(ng * n, DD), atab)
    h, atab = _block_feat_tc(
        ng, n, convraw, dpart, s128,
        _stack(params, l1, 'gat_bias').reshape(ng, 1, DD),
        _stack(params, l1, 'pro_W'),
        _stack(params, l1, 'sa_Wq'), _stack(params, l1, 'sa_Wk'),
        _stack(params, l1, 'ln_g').reshape(ng, 1, DD),
        _stack(params, l1, 'ln_b').reshape(ng, 1, DD),
        _stack(params, l2, 'W'),
        _stack(params, l2, 'att_src').reshape(ng, 1, DD),
        _stack(params, l2, 'att_dst').reshape(ng, 1, DD), s8s, s8d)
    convraw, dpart = gat_sc(edges, h.reshape(ng * n, DD), atab)
    x = _block_tc(ng, n, convraw, dpart, s128,
                  _stack(params, l2, 'gat_bias').reshape(ng, 1, DD),
                  _stack(params, l2, 'pro_W'),
                  _stack(params, l2, 'sa_Wq'), _stack(params, l2, 'sa_Wk'),
                  _stack(params, l2, 'ln_g').reshape(ng, 1, DD),
                  _stack(params, l2, 'ln_b').reshape(ng, 1, DD))
    pooled, cpart = _make_pool_sc(ng, n)(x, node_t)
    return _poolnorm_tc(ng, n, pooled, cpart)


def kernel(cell_adj_matrix, cell_feat_matrix, cell_node_set, drug_adj_matrix,
           drug_feat_matrix, drug_node_set, idx_cell_drug, params):
    ncell, fcell = cell_feat_matrix.shape[2], cell_feat_matrix.shape[3]
    ndrug, fdrug = drug_feat_matrix.shape[2], drug_feat_matrix.shape[3]
    ecell = cell_adj_matrix.shape[3]
    edrug = drug_adj_matrix.shape[3]
    npair = idx_cell_drug.shape[0]

    cell_edges = cell_adj_matrix.reshape(6, 2, ecell).astype(I32)
    drug_edges = drug_adj_matrix.reshape(6, 2, edrug).astype(I32)
    cell_node_t = jnp.transpose(
        cell_node_set.reshape(6, ncell, KNODE), (0, 2, 1)).astype(I32)
    drug_node_t = jnp.transpose(
        drug_node_set.reshape(6, ndrug, KNODE), (0, 2, 1)).astype(I32)

    oc = _graph_tower(cell_feat_matrix.reshape(6, ncell, fcell), cell_edges,
                      cell_node_t, params, 'cell', ncell, ecell, fcell)
    od = _graph_tower(drug_feat_matrix.reshape(6, ndrug, fdrug), drug_edges,
                      drug_node_t, params, 'drug', ndrug, edrug, fdrug)

    dcol = 6 * DD
    idxt = jnp.transpose(idx_cell_drug, (1, 0)).astype(I32)
    gc, gd = _make_pair_sc(npair, dcol)(oc, od, idxt)
    out = _mlp_tc(npair, dcol, gc, gd,
                  params['fc1_W'][:dcol], params['fc1_W'][dcol:],
                  params['fc1_b'].reshape(1, 256),
                  params['fc2_W'], params['fc2_b'].reshape(1, 256),
                  params['fc3_W'], params['fc3_b'].reshape(1, 1))
    return out.reshape(npair)


# speedup vs baseline: 1.2729x; 1.1196x over previous
import functools

import jax
import jax.numpy as jnp
from jax import lax
from jax.experimental import pallas as pl
from jax.experimental.pallas import tpu as pltpu
from jax.experimental.pallas import tpu_sc as plsc

F32 = jnp.float32
I32 = jnp.int32
NHEADS = 4
NHID = 32
DD = NHEADS * NHID
KNODE = 4
NCORE = 2
NSUB = 16
NWORK = NCORE * NSUB


def _feat_body(x_ref, w_ref, asrc_ref, adst_ref, s8s_ref, s8d_ref, h_ref, atab_ref):
    x = x_ref[0]
    w = w_ref[0]
    h = jnp.dot(x, w, preferred_element_type=F32)
    h_ref[0] = h
    asrc = asrc_ref[0]
    adst = adst_ref[0]
    atab_ref[0] = (lax.dot_general(s8s_ref[...], h * asrc,
                                   (((0,), (1,)), ((), ())),
                                   preferred_element_type=F32)
                   + lax.dot_general(s8d_ref[...], h * adst,
                                     (((0,), (1,)), ((), ())),
                                     preferred_element_type=F32))


def _feat_tc(ng, n, fin, x, w, asrc, adst, s8s, s8d):
    return pl.pallas_call(
        _feat_body,
        grid=(ng,),
        in_specs=[
            pl.BlockSpec((1, n, fin), lambda g: (g, 0, 0)),
            pl.BlockSpec((1, fin, DD), lambda g: (g, 0, 0)),
            pl.BlockSpec((1, 1, DD), lambda g: (g, 0, 0)),
            pl.BlockSpec((1, 1, DD), lambda g: (g, 0, 0)),
            pl.BlockSpec((DD, 8), lambda g: (0, 0)),
            pl.BlockSpec((DD, 8), lambda g: (0, 0)),
        ],
        out_specs=[
            pl.BlockSpec((1, n, DD), lambda g: (g, 0, 0)),
            pl.BlockSpec((1, 8, n), lambda g: (g, 0, 0)),
        ],
        out_shape=[
            jax.ShapeDtypeStruct((ng, n, DD), F32),
            jax.ShapeDtypeStruct((ng, 8, n), F32),
        ],
    )(x, w, asrc, adst, s8s, s8d)


def _make_gat_sc(ng, n, e):
    ew = e // NWORK
    nch = ew // 128
    nvr = ew // 16
    n16 = n // NSUB
    mesh = plsc.VectorSubcoreMesh(core_axis_name="c", subcore_axis_name="s")

    @functools.partial(
        pl.kernel,
        mesh=mesh,
        compiler_params=pltpu.CompilerParams(needs_layout_passes=False),
        out_type=[
            jax.ShapeDtypeStruct((ng, NCORE, n, DD), F32),
            jax.ShapeDtypeStruct((ng, NWORK * NHEADS, n), F32),
        ],
        scratch_types=[
            pltpu.VMEM((ng, ew), I32),
            pltpu.VMEM((ng, ew), I32),
            pltpu.VMEM((2, 8, n), F32),
            pltpu.VMEM((NHEADS * ew,), F32),
            pltpu.VMEM((NHEADS, n), F32),
            pltpu.VMEM((nch, 128), I32),
            pltpu.VMEM((nch, 128), I32),
            pltpu.VMEM((ew, DD), F32),
            pltpu.VMEM((n16, DD), F32),
            pltpu.VMEM_SHARED((n, DD), F32),
            pltpu.SemaphoreType.DMA,
            pltpu.SemaphoreType.DMA,
            pltpu.SemaphoreType.DMA,
            pltpu.SemaphoreType.DMA,
        ],
    )
    def gat(edges_hbm, h_hbm, atab_hbm, convraw_hbm, dpart_hbm,
            src_v, dst_v, atab_v, ee_v, den_v, gidx_v, didx_v, rows_v,
            zero_v, shout, gsem, ssem, asem, dsem):
        cid = lax.axis_index("c")
        sid = lax.axis_index("s")
        wid = cid * NSUB + sid
        base = wid * ew
        zv16 = jnp.zeros((16,), F32)

        def zrow(r, _):
            for c in range(DD // 16):
                zero_v[r, pl.ds(c * 16, 16)] = zv16
            return 0
        lax.fori_loop(0, n16, zrow, 0)

        pltpu.sync_copy(edges_hbm.at[:, 0, pl.ds(base, ew)], src_v)
        pltpu.sync_copy(edges_hbm.at[:, 1, pl.ds(base, ew)], dst_v)
        pltpu.sync_copy(atab_hbm.at[0], atab_v.at[0])

        def gbody(g, _):
            p = lax.rem(g, 2)
            pltpu.async_copy(atab_hbm.at[jnp.minimum(g + 1, ng - 1)],
                             atab_v.at[1 - p], asem)

            @plsc.parallel_loop(0, nvr, step=1, unroll=4)
            def _ibody(i):
                srcv = src_v[g, pl.ds(i * 16, 16)]
                gidx_v[i // 8, pl.ds((i % 8) * 16, 16)] = srcv + g * n
                didx_v[i // 8, pl.ds((i % 8) * 16, 16)] = \
                    dst_v[g, pl.ds(i * 16, 16)]

            gdesc = [pltpu.async_copy(h_hbm.at[gidx_v.at[j]],
                                      rows_v.at[pl.ds(j * 128, 128)], gsem)
                     for j in range(nch)]

            pltpu.sync_copy(zero_v, shout.at[pl.ds(sid * n16, n16)])

            @plsc.parallel_loop(0, n // 16, step=1, unroll=4)
            def _dzero(i):
                for hd in range(NHEADS):
                    den_v[hd, pl.ds(i * 16, 16)] = zv16

            p16 = jnp.zeros((16,), I32) + p

            def ebody(i, _):
                srcv = src_v[g, pl.ds(i * 16, 16)]
                dstv = dst_v[g, pl.ds(i * 16, 16)]
                for hd in range(NHEADS):
                    h16 = jnp.full((16,), hd, I32)
                    a_s = plsc.load_gather(atab_v, [p16, h16, srcv])
                    a_d = plsc.load_gather(atab_v, [p16, h16 + 4, dstv])
                    ev = a_s + a_d
                    ev = jnp.where(ev >= 0.0, ev, ev * 0.2)
                    eev = jnp.exp(ev)
                    ee_v[pl.ds(hd * ew + i * 16, 16)] = eev
                    plsc.addupdate_scatter(den_v, [h16, dstv], eev)
                return 0
            lax.fori_loop(0, nvr, ebody, 0)

            ddesc = pltpu.async_copy(
                den_v, dpart_hbm.at[g, pl.ds(wid * NHEADS, NHEADS)], dsem)
            plsc.subcore_barrier()

            sdesc = []
            for j in range(nch):
                gdesc[j].wait()

                @plsc.parallel_loop(j * 128, (j + 1) * 128, step=1, unroll=4)
                def _mbody(eidx):
                    ev16 = jnp.zeros((16,), I32) + eidx
                    for hd in range(NHEADS):
                        s = plsc.load_gather(ee_v, [ev16 + hd * ew])
                        for cc in range(2):
                            c0 = (hd * 2 + cc) * 16
                            rows_v[eidx, pl.ds(c0, 16)] = \
                                rows_v[eidx, pl.ds(c0, 16)] * s

                sdesc.append(
                    pltpu.async_copy(rows_v.at[pl.ds(j * 128, 128)],
                                     shout.at[didx_v.at[j]], ssem, add=True))

            for d in sdesc:
                d.wait()
            plsc.subcore_barrier()
            pltpu.sync_copy(shout.at[pl.ds(sid * n16, n16)],
                            convraw_hbm.at[g, cid, pl.ds(sid * n16, n16)])
            ddesc.wait()
            pltpu.make_async_copy(atab_hbm.at[0], atab_v.at[0], asem).wait()
            return 0
        lax.fori_loop(0, ng, gbody, 0)

    return gat


def _block_core(convraw_ref, dpart_ref, s128_ref, bias_ref, prow_ref,
                wq_ref, wk_ref, lng_ref, lnb_ref):
    raw = convraw_ref[0, 0] + convraw_ref[0, 1]
    dsum = dpart_ref[0, pl.ds(0, 2 * NHEADS)]
    for i in range(1, NWORK // 2):
        dsum = dsum + dpart_ref[0, pl.ds(i * 2 * NHEADS, 2 * NHEADS)]
    denom = lax.dot_general(dsum, s128_ref[...],
                            (((0,), (1,)), ((), ())),
                            preferred_element_type=F32)
    conv = raw / (denom + 1e-16) + bias_ref[0]
    h2 = jnp.dot(conv, prow_ref[0], preferred_element_type=F32)
    temp = jnp.zeros_like(h2)
    for hd in range(NHEADS):
        q = jnp.dot(h2, wq_ref[0, hd], preferred_element_type=F32)
        k = jnp.dot(h2, wk_ref[0, hd], preferred_element_type=F32)
        s = lax.dot_general(q, k, (((1,), (1,)), ((), ())),
                            preferred_element_type=F32) * (1.0 / jnp.sqrt(32.0))
        p = jnp.exp(s)
        r = jnp.dot(p, h2, preferred_element_type=F32)
        temp = temp + r / jnp.sum(p, axis=-1, keepdims=True)
    y = temp + h2
    m = jnp.mean(y, axis=-1, keepdims=True)
    v = jnp.mean((y - m) * (y - m), axis=-1, keepdims=True)
    return lng_ref[0] * (y - m) / jnp.sqrt(v + 1e-5) + lnb_ref[0]


def _block_body(convraw_ref, dpart_ref, s128_ref, bias_ref, prow_ref,
                wq_ref, wk_ref, lng_ref, lnb_ref, xout_ref):
    xout_ref[0] = _block_core(convraw_ref, dpart_ref, s128_ref, bias_ref,
                              prow_ref, wq_ref, wk_ref, lng_ref, lnb_ref)


def _block_feat_body(convraw_ref, dpart_ref, s128_ref, bias_ref, prow_ref,
                     wq_ref, wk_ref, lng_ref, lnb_ref, w2_ref, asrc2_ref,
                     adst2_ref, s8s_ref, s8d_ref, h_ref, atab_ref):
    x = _block_core(convraw_ref, dpart_ref, s128_ref, bias_ref,
                    prow_ref, wq_ref, wk_ref, lng_ref, lnb_ref)
    h = jnp.dot(x, w2_ref[0], preferred_element_type=F32)
    h_ref[0] = h
    atab_ref[0] = (lax.dot_general(s8s_ref[...], h * asrc2_ref[0],
                                   (((0,), (1,)), ((), ())),
                                   preferred_element_type=F32)
                   + lax.dot_general(s8d_ref[...], h * adst2_ref[0],
                                     (((0,), (1,)), ((), ())),
                                     preferred_element_type=F32))


def _block_in_specs(n):
    return [
        pl.BlockSpec((1, NCORE, n, DD), lambda g: (g, 0, 0, 0)),
        pl.BlockSpec((1, NWORK * NHEADS, n), lambda g: (g, 0, 0)),
        pl.BlockSpec((DD, 8), lambda g: (0, 0)),
        pl.BlockSpec((1, 1, DD), lambda g: (g, 0, 0)),
        pl.BlockSpec((1, DD, DD), lambda g: (g, 0, 0)),
        pl.BlockSpec((1, NHEADS, DD, NHID), lambda g: (g, 0, 0, 0)),
        pl.BlockSpec((1, NHEADS, DD, NHID), lambda g: (g, 0, 0, 0)),
        pl.BlockSpec((1, 1, DD), lambda g: (g, 0, 0)),
        pl.BlockSpec((1, 1, DD), lambda g: (g, 0, 0)),
    ]


def _block_tc(ng, n, convraw, dpart, s128, bias, prow, wq, wk, lng, lnb):
    return pl.pallas_call(
        _block_body,
        grid=(ng,),
        in_specs=_block_in_specs(n),
        out_specs=pl.BlockSpec((1, n, DD), lambda g: (g, 0, 0)),
        out_shape=jax.ShapeDtypeStruct((ng, n, DD), F32),
    )(convraw, dpart, s128, bias, prow, wq, wk, lng, lnb)


def _block_feat_tc(ng, n, convraw, dpart, s128, bias, prow, wq, wk, lng, lnb,
                   w2, asrc2, adst2, s8s, s8d):
    return pl.pallas_call(
        _block_feat_body,
        grid=(ng,),
        in_specs=_block_in_specs(n) + [
            pl.BlockSpec((1, DD, DD), lambda g: (g, 0, 0)),
            pl.BlockSpec((1, 1, DD), lambda g: (g, 0, 0)),
            pl.BlockSpec((1, 1, DD), lambda g: (g, 0, 0)),
            pl.BlockSpec((DD, 8), lambda g: (0, 0)),
            pl.BlockSpec((DD, 8), lambda g: (0, 0)),
        ],
        out_specs=[
            pl.BlockSpec((1, n, DD), lambda g: (g, 0, 0)),
            pl.BlockSpec((1, 8, n), lambda g: (g, 0, 0)),
        ],
        out_shape=[
            jax.ShapeDtypeStruct((ng, n, DD), F32),
            jax.ShapeDtypeStruct((ng, 8, n), F32),
        ],
    )(convraw, dpart, s128, bias, prow, wq, wk, lng, lnb,
      w2, asrc2, adst2, s8s, s8d)


def _make_pool_sc(ng, n):
    nw = n // NWORK
    n16 = n // NSUB
    mesh = plsc.VectorSubcoreMesh(core_axis_name="c", subcore_axis_name="s")

    @functools.partial(
        pl.kernel,
        mesh=mesh,
        compiler_params=pltpu.CompilerParams(needs_layout_passes=False),
        out_type=[
            jax.ShapeDtypeStruct((ng, NCORE, n, DD), F32),
            jax.ShapeDtypeStruct((ng, NWORK, n), F32),
        ],
        scratch_types=[
            pltpu.VMEM((nw, DD), F32),
            pltpu.VMEM((KNODE, nw), I32),
            pltpu.VMEM((n,), F32),
            pltpu.VMEM((n16, DD), F32),
            pltpu.VMEM_SHARED((n, DD), F32),
            pltpu.SemaphoreType.DMA,
        ],
    )
    def pool(x_hbm, node_hbm, pooled_hbm, cpart_hbm,
             rows_v, idx_v, cnt_v, zero_v, shpool, sem):
        cid = lax.axis_index("c")
        sid = lax.axis_index("s")
        wid = cid * NSUB + sid
        base = wid * nw
        zv16 = jnp.zeros((16,), F32)
        ones16 = jnp.full((16,), 1.0, F32)

        def zrow(r, _):
            for c in range(DD // 16):
                zero_v[r, pl.ds(c * 16, 16)] = zv16
            return 0
        lax.fori_loop(0, n16, zrow, 0)

        def gbody(g, _):
            pltpu.sync_copy(x_hbm.at[g, pl.ds(base, nw)], rows_v)
            for k in range(KNODE):
                pltpu.sync_copy(node_hbm.at[g, k, pl.ds(base, nw)], idx_v.at[k])
            pltpu.sync_copy(zero_v, shpool.at[pl.ds(sid * n16, n16)])

            def czero(i, _):
                cnt_v[pl.ds(i * 16, 16)] = zv16
                return 0
            lax.fori_loop(0, n // 16, czero, 0)

            for k in range(KNODE):
                for c in range(nw // 16):
                    idxv = idx_v[k, pl.ds(c * 16, 16)]
                    plsc.addupdate_scatter(cnt_v, [idxv], ones16)
            pltpu.sync_copy(cnt_v, cpart_hbm.at[g, wid])
            plsc.subcore_barrier()
            sdesc = [pltpu.async_copy(rows_v, shpool.at[idx_v.at[k]],
                                      sem, add=True)
                     for k in range(KNODE)]
            for d in sdesc:
                d.wait()
            plsc.subcore_barrier()
            pltpu.sync_copy(shpool.at[pl.ds(sid * n16, n16)],
                            pooled_hbm.at[g, cid, pl.ds(sid * n16, n16)])
            return 0
        lax.fori_loop(0, ng, gbody, 0)

    return pool


def _poolnorm_body(pooled_ref, cpart_ref, out_ref):
    pool = pooled_ref[0, 0] + pooled_ref[0, 1]
    cnt2d = cpart_ref[0]
    ones = jnp.ones((NWORK, 1), F32)
    cnt = lax.dot_general(cnt2d, ones, (((0,), (0,)), ((), ())),
                          preferred_element_type=F32)
    out_ref[...] = pool / jnp.maximum(cnt, 1.0)


def _poolnorm_tc(ng, n, pooled, cpart):
    return pl.pallas_call(
        _poolnorm_body,
        grid=(ng,),
        in_specs=[
            pl.BlockSpec((1, NCORE, n, DD), lambda g: (g, 0, 0, 0)),
            pl.BlockSpec((1, NWORK, n), lambda g: (g, 0, 0)),
        ],
        out_specs=pl.BlockSpec((n, DD), lambda g: (0, g)),
        out_shape=jax.ShapeDtypeStruct((n, ng * DD), F32),
    )(pooled, cpart)


def _make_pair_sc(npair, dcol):
    pw = npair // NWORK
    mesh = plsc.VectorSubcoreMesh(core_axis_name="c", subcore_axis_name="s")

    @functools.partial(
        pl.kernel,
        mesh=mesh,
        compiler_params=pltpu.CompilerParams(needs_layout_passes=False),
        out_type=[
            jax.ShapeDtypeStruct((npair, dcol), F32),
            jax.ShapeDtypeStruct((npair, dcol), F32),
        ],
        scratch_types=[
            pltpu.VMEM((2, pw), I32),
            pltpu.VMEM((pw, dcol), F32),
            pltpu.SemaphoreType.DMA,
        ],
    )
    def pair(oc_hbm, od_hbm, idxt_hbm, gc_hbm, gd_hbm, pidx_v, rows_v, sem):
        cid = lax.axis_index("c")
        sid = lax.axis_index("s")
        wid = cid * NSUB + sid
        base = wid * pw
        pltpu.sync_copy(idxt_hbm.at[0, pl.ds(base, pw)], pidx_v.at[0])
        pltpu.sync_copy(idxt_hbm.at[1, pl.ds(base, pw)], pidx_v.at[1])
        pltpu.async_copy(oc_hbm.at[pidx_v.at[0]], rows_v, sem).wait()
        pltpu.sync_copy(rows_v, gc_hbm.at[pl.ds(base, pw)])
        pltpu.async_copy(od_hbm.at[pidx_v.at[1]], rows_v, sem).wait()
        pltpu.sync_copy(rows_v, gd_hbm.at[pl.ds(base, pw)])

    return pair


def _mlp_body(gc_ref, gd_ref, w1c_ref, w1d_ref, b1_ref, w2_ref, b2_ref,
              w3_ref, b3_ref, out_ref):
    x = (jnp.dot(gc_ref[...], w1c_ref[...], preferred_element_type=F32)
         + jnp.dot(gd_ref[...], w1d_ref[...], preferred_element_type=F32)
         + b1_ref[...])
    x = jnp.maximum(x, 0.0)
    x = jnp.dot(x, w2_ref[...], preferred_element_type=F32) + b2_ref[...]
    x = jnp.maximum(x, 0.0)
    out_ref[...] = jnp.dot(x, w3_ref[...], preferred_element_type=F32) \
        + b3_ref[...]


def _mlp_tc(npair, dcol, gc, gd, w1c, w1d, b1, w2, b2, w3, b3):
    bp = 512
    return pl.pallas_call(
        _mlp_body,
        grid=(npair // bp,),
        in_specs=[
            pl.BlockSpec((bp, dcol), lambda g: (g, 0)),
            pl.BlockSpec((bp, dcol), lambda g: (g, 0)),
            pl.BlockSpec((dcol, 256), lambda g: (0, 0)),
            pl.BlockSpec((dcol, 256), lambda g: (0, 0)),
            pl.BlockSpec((1, 256), lambda g: (0, 0)),
            pl.BlockSpec((256, 256), lambda g: (0, 0)),
            pl.BlockSpec((1, 256), lambda g: (0, 0)),
            pl.BlockSpec((256, 1), lambda g: (0, 0)),
            pl.BlockSpec((1, 1), lambda g: (0, 0)),
        ],
        out_specs=pl.BlockSpec((bp, 1), lambda g: (g, 0)),
        out_shape=jax.ShapeDtypeStruct((npair, 1), F32),
    )(gc, gd, w1c, w1d, b1, w2, b2, w3, b3)


def _stack(params, prefix, name):
    return jnp.stack([params['%s_%d_%d' % (prefix, i, j)][name]
                      for i in range(2) for j in range(3)])


def _graph_tower(feat, edges, node_t, params, prefix, n, e, fin):
    ng = 6
    s8s = (jnp.arange(DD)[:, None] // NHID ==
           jnp.arange(8)[None, :]).astype(F32)
    s8d = (jnp.arange(DD)[:, None] // NHID ==
           (jnp.arange(8)[None, :] - 4)).astype(F32)
    s128 = s8s + s8d
    gat_sc = _make_gat_sc(ng, n, e)
    l1, l2 = prefix, prefix + '2'
    h, atab = _feat_tc(
        ng, n, fin, feat, _stack(params, l1, 'W'),
        _stack(params, l1, 'att_src').reshape(ng, 1, DD),
        _stack(params, l1, 'att_dst').reshape(ng, 1, DD), s8s, s8d)
    convraw, dpart = gat_sc(edges, h.reshape(ng * n, DD), atab)
    h, atab = _block_feat_tc(
        ng, n, convraw, dpart, s128,
        _stack(params, l1, 'gat_bias').reshape(ng, 1, DD),
        _stack(params, l1, 'pro_W'),
        _stack(params, l1, 'sa_Wq'), _stack(params, l1, 'sa_Wk'),
        _stack(params, l1, 'ln_g').reshape(ng, 1, DD),
        _stack(params, l1, 'ln_b').reshape(ng, 1, DD),
        _stack(params, l2, 'W'),
        _stack(params, l2, 'att_src').reshape(ng, 1, DD),
        _stack(params, l2, 'att_dst').reshape(ng, 1, DD), s8s, s8d)
    convraw, dpart = gat_sc(edges, h.reshape(ng * n, DD), atab)
    x = _block_tc(ng, n, convraw, dpart, s128,
                  _stack(params, l2, 'gat_bias').reshape(ng, 1, DD),
                  _stack(params, l2, 'pro_W'),
                  _stack(params, l2, 'sa_Wq'), _stack(params, l2, 'sa_Wk'),
                  _stack(params, l2, 'ln_g').reshape(ng, 1, DD),
                  _stack(params, l2, 'ln_b').reshape(ng, 1, DD))
    pooled, cpart = _make_pool_sc(ng, n)(x, node_t)
    return _poolnorm_tc(ng, n, pooled, cpart)


def kernel(cell_adj_matrix, cell_feat_matrix, cell_node_set, drug_adj_matrix,
           drug_feat_matrix, drug_node_set, idx_cell_drug, params):
    ncell, fcell = cell_feat_matrix.shape[2], cell_feat_matrix.shape[3]
    ndrug, fdrug = drug_feat_matrix.shape[2], drug_feat_matrix.shape[3]
    ecell = cell_adj_matrix.shape[3]
    edrug = drug_adj_matrix.shape[3]
    npair = idx_cell_drug.shape[0]

    cell_edges = cell_adj_matrix.reshape(6, 2, ecell).astype(I32)
    drug_edges = drug_adj_matrix.reshape(6, 2, edrug).astype(I32)
    cell_node_t = jnp.transpose(
        cell_node_set.reshape(6, ncell, KNODE), (0, 2, 1)).astype(I32)
    drug_node_t = jnp.transpose(
        drug_node_set.reshape(6, ndrug, KNODE), (0, 2, 1)).astype(I32)

    oc = _graph_tower(cell_feat_matrix.reshape(6, ncell, fcell), cell_edges,
                      cell_node_t, params, 'cell', ncell, ecell, fcell)
    od = _graph_tower(drug_feat_matrix.reshape(6, ndrug, fdrug), drug_edges,
                      drug_node_t, params, 'drug', ndrug, edrug, fdrug)

    dcol = 6 * DD
    idxt = jnp.transpose(idx_cell_drug, (1, 0)).astype(I32)
    gc, gd = _make_pair_sc(npair, dcol)(oc, od, idxt)
    out = _mlp_tc(npair, dcol, gc, gd,
                  params['fc1_W'][:dcol], params['fc1_W'][dcol:],
                  params['fc1_b'].reshape(1, 256),
                  params['fc2_W'], params['fc2_b'].reshape(1, 256),
                  params['fc3_W'], params['fc3_b'].reshape(1, 1))
    return out.reshape(npair)
